# baseline (speedup 1.0000x reference)
import functools

import jax
import jax.numpy as jnp
from jax import lax
from jax.experimental import pallas as pl
from jax.experimental.pallas import tpu as pltpu
from jax.experimental.pallas import tpu_sc as plsc

N = 10000
E = 320000
D_IN = 128
D = 16
NC, NS = 2, 16
NW = NC * NS
BLK = 128
JPW = 80
KB = 8
NG = JPW // KB
EP = NW * JPW * BLK
NP = N + 112
RPS = 632
RPS_L = N - (NS - 1) * RPS
RB = 2
RBS = N // RB

def _zero_acc(zeros_hbm, stage_v, acc, sid):
    pltpu.sync_copy(zeros_hbm, stage_v)

    @pl.when(sid < NS - 1)
    def _():
        pltpu.sync_copy(stage_v, acc.at[pl.ds(sid * RPS, RPS)])

    @pl.when(sid == NS - 1)
    def _():
        pltpu.sync_copy(stage_v.at[pl.ds(0, RPS_L)],
                        acc.at[pl.ds(sid * RPS, RPS_L)])


def _copy_out(out_hbm, stage_v, acc, cid, sid):
    @pl.when(sid < NS - 1)
    def _():
        pltpu.sync_copy(acc.at[pl.ds(sid * RPS, RPS)], stage_v)
        pltpu.sync_copy(stage_v, out_hbm.at[cid, pl.ds(sid * RPS, RPS)])

    @pl.when(sid == NS - 1)
    def _():
        pltpu.sync_copy(acc.at[pl.ds(sid * RPS, RPS_L)],
                        stage_v.at[pl.ds(0, RPS_L)])
        pltpu.sync_copy(stage_v.at[pl.ds(0, RPS_L)],
                        out_hbm.at[cid, pl.ds(sid * RPS, RPS_L)])


def _sc_deg_body(dst_hbm, ones_hbm, zeros_hbm, out_hbm, idx_v, val_v, stage_v,
                 sem, acc):
    cid = lax.axis_index("c")
    sid = lax.axis_index("s")
    wid = cid * NS + sid
    _zero_acc(zeros_hbm, stage_v, acc, sid)
    pltpu.sync_copy(dst_hbm.at[wid], idx_v)
    pltpu.sync_copy(ones_hbm, val_v)
    plsc.subcore_barrier()

    def body(j, carry):
        pltpu.sync_copy(val_v, acc.at[idx_v.at[j]], add=True)
        return carry

    lax.fori_loop(0, JPW, body, 0)
    plsc.subcore_barrier()
    _copy_out(out_hbm, stage_v, acc, cid, sid)


@functools.cache
def _sc_kernels():
    mesh = plsc.VectorSubcoreMesh(core_axis_name="c", subcore_axis_name="s",
                                  num_cores=NC, num_subcores=NS)
    params = pltpu.CompilerParams(use_tc_tiling_on_sc=False)
    sc_deg = pl.kernel(
        _sc_deg_body,
        out_type=jax.ShapeDtypeStruct((NC, N, D), jnp.float32),
        mesh=mesh,
        scratch_types=[
            pltpu.VMEM((JPW, BLK), jnp.int32),
            pltpu.VMEM((BLK, D), jnp.float32),
            pltpu.VMEM((RPS, D), jnp.float32),
            pltpu.SemaphoreType.DMA,
            pltpu.VMEM_SHARED((NP, D), jnp.float32),
        ],
        compiler_params=params,
    )
    sc_agg = pl.kernel(
        _sc_agg_body,
        out_type=jax.ShapeDtypeStruct((NC, N, D), jnp.float32),
        mesh=mesh,
        scratch_types=[
            pltpu.VMEM((JPW, BLK), jnp.int32),
            pltpu.VMEM((JPW, BLK), jnp.int32),
            pltpu.VMEM((2, KB, BLK, D), jnp.float32),
            pltpu.VMEM((RPS, D), jnp.float32),
            pltpu.SemaphoreType.DMA,
            pltpu.VMEM_SHARED((NP, D), jnp.float32),
        ],
        compiler_params=params,
    )
    return sc_deg, sc_agg


def _sc_agg_body(src_hbm, dst_hbm, hs_hbm, zeros_hbm, out_hbm, sidx_v, didx_v,
                 rows_v, stage_v, sem, acc):
    cid = lax.axis_index("c")
    sid = lax.axis_index("s")
    wid = cid * NS + sid
    _zero_acc(zeros_hbm, stage_v, acc, sid)
    pltpu.sync_copy(src_hbm.at[wid], sidx_v)
    pltpu.sync_copy(dst_hbm.at[wid], didx_v)
    plsc.subcore_barrier()

    def fire(g, p):
        for t in range(KB):
            pltpu.async_copy(hs_hbm.at[sidx_v.at[g * KB + t]],
                             rows_v.at[p, t], sem)

    fire(0, 0)

    def body(g, carry):
        p = lax.rem(g, 2)

        @pl.when(g < NG - 1)
        def _():
            fire(g + 1, 1 - p)

        pltpu.make_async_copy(hs_hbm.at[pl.ds(0, KB * BLK)],
                              rows_v.at[p], sem).wait()
        for t in range(KB):
            pltpu.sync_copy(rows_v.at[p, t],
                            acc.at[didx_v.at[g * KB + t]], add=True)
        return carry

    lax.fori_loop(0, NG, body, 0)
    plsc.subcore_barrier()
    _copy_out(out_hbm, stage_v, acc, cid, sid)


def _tc_mm_body(x_ref, w_ref, o_ref):
    o_ref[...] = jnp.dot(x_ref[...], w_ref[...],
                         preferred_element_type=jnp.float32)


def _tc_scale_body(d0_ref, d1_ref, h_ref, hs_ref, dw_ref):
    deg = d0_ref[0, :, 0:1] + d1_ref[0, :, 0:1] + 1.0
    dis = lax.rsqrt(deg)
    dw_ref[...] = jnp.broadcast_to(dis, h_ref.shape)
    hs_ref[...] = dis * h_ref[...]


def _tc_mid_body(a0_ref, a1_ref, hs_ref, dw_ref, b_ref, w_ref, o_ref):
    dw = dw_ref[...]
    z = dw * (a0_ref[0] + a1_ref[0] + hs_ref[...]) + b_ref[...]
    z = jnp.maximum(z, 0.0)
    o_ref[...] = dw * jnp.dot(z, w_ref[...],
                              preferred_element_type=jnp.float32)


def _tc_fin_body(a0_ref, a1_ref, hs_ref, dw_ref, b_ref, o_ref):
    o_ref[...] = (dw_ref[...] * (a0_ref[0] + a1_ref[0] + hs_ref[...])
                  + b_ref[...])


def _row_spec(width):
    return pl.BlockSpec((RBS, width), lambda i: (i, 0))


def _part_spec(core):
    return pl.BlockSpec((1, RBS, D), lambda i, c=core: (c, i, 0))


def _full_spec(shape):
    return pl.BlockSpec(shape, lambda i: (0,) * len(shape))


def _nd_out():
    return jax.ShapeDtypeStruct((N, D), jnp.float32)


def kernel(x, edge_index, W1, b1, W2, b2):
    sc_deg, sc_agg = _sc_kernels()
    ei = edge_index.astype(jnp.int32)
    pad = EP - E
    src_r = jnp.concatenate(
        [ei[0], jnp.zeros((pad,), jnp.int32)]).reshape(NW, JPW, BLK)
    pad_dst = N + (jnp.arange(pad, dtype=jnp.int32) % (NP - N))
    dst_r = jnp.concatenate([ei[1], pad_dst]).reshape(NW, JPW, BLK)
    ones_rows = jnp.zeros((BLK, D), jnp.float32).at[:, 0].set(1.0)
    zeros_blk = jnp.zeros((RPS, D), jnp.float32)

    h1 = pl.pallas_call(
        _tc_mm_body,
        grid=(RB,),
        in_specs=[_row_spec(D_IN), _full_spec((D_IN, D))],
        out_specs=_row_spec(D),
        out_shape=_nd_out(),
    )(x, W1)

    degp = sc_deg(dst_r, ones_rows, zeros_blk)

    hs1, dw = pl.pallas_call(
        _tc_scale_body,
        grid=(RB,),
        in_specs=[_part_spec(0), _part_spec(1), _row_spec(D)],
        out_specs=[_row_spec(D), _row_spec(D)],
        out_shape=[_nd_out(), _nd_out()],
    )(degp, degp, h1)

    acc1 = sc_agg(src_r, dst_r, hs1, zeros_blk)

    hs2 = pl.pallas_call(
        _tc_mid_body,
        grid=(RB,),
        in_specs=[_part_spec(0), _part_spec(1), _row_spec(D), _row_spec(D),
                  _full_spec((1, D)), _full_spec((D, D))],
        out_specs=_row_spec(D),
        out_shape=_nd_out(),
    )(acc1, acc1, hs1, dw, b1.reshape(1, D), W2)

    acc2 = sc_agg(src_r, dst_r, hs2, zeros_blk)

    out = pl.pallas_call(
        _tc_fin_body,
        grid=(RB,),
        in_specs=[_part_spec(0), _part_spec(1), _row_spec(D), _row_spec(D),
                  _full_spec((1, D))],
        out_specs=_row_spec(D),
        out_shape=_nd_out(),
    )(acc2, acc2, hs2, dw, b2.reshape(1, D))
    return out


# speedup vs baseline: 1.0143x; 1.0143x over previous
import functools

import jax
import jax.numpy as jnp
from jax import lax
from jax.experimental import pallas as pl
from jax.experimental.pallas import tpu as pltpu
from jax.experimental.pallas import tpu_sc as plsc

N = 10000
E = 320000
D_IN = 128
D = 16
NC, NS = 2, 16
NW = NC * NS
BLK = 128
JPW = 80
KB = 8
NG = JPW // KB
EP = NW * JPW * BLK
NP = N + 112
RPS = 632
RPS_L = N - (NS - 1) * RPS
RB = 2
RBS = N // RB

def _zero_acc(zeros_hbm, stage_v, acc, sid):
    pltpu.sync_copy(zeros_hbm, stage_v)

    @pl.when(sid < NS - 1)
    def _():
        pltpu.sync_copy(stage_v, acc.at[pl.ds(sid * RPS, RPS)])

    @pl.when(sid == NS - 1)
    def _():
        pltpu.sync_copy(stage_v.at[pl.ds(0, RPS_L)],
                        acc.at[pl.ds(sid * RPS, RPS_L)])


def _copy_out(out_hbm, stage_v, acc, cid, sid):
    @pl.when(sid < NS - 1)
    def _():
        pltpu.sync_copy(acc.at[pl.ds(sid * RPS, RPS)], stage_v)
        pltpu.sync_copy(stage_v, out_hbm.at[cid, pl.ds(sid * RPS, RPS)])

    @pl.when(sid == NS - 1)
    def _():
        pltpu.sync_copy(acc.at[pl.ds(sid * RPS, RPS_L)],
                        stage_v.at[pl.ds(0, RPS_L)])
        pltpu.sync_copy(stage_v.at[pl.ds(0, RPS_L)],
                        out_hbm.at[cid, pl.ds(sid * RPS, RPS_L)])


def _sc_deg_body(dst_hbm, ones_hbm, zeros_hbm, out_hbm, idx_v, val_v, stage_v,
                 sem, acc):
    cid = lax.axis_index("c")
    sid = lax.axis_index("s")
    wid = cid * NS + sid
    _zero_acc(zeros_hbm, stage_v, acc, sid)
    pltpu.sync_copy(dst_hbm.at[wid], idx_v)
    pltpu.sync_copy(ones_hbm, val_v)
    plsc.subcore_barrier()

    def body(j, carry):
        pltpu.sync_copy(val_v, acc.at[idx_v.at[j]], add=True)
        return carry

    lax.fori_loop(0, JPW, body, 0)
    plsc.subcore_barrier()
    _copy_out(out_hbm, stage_v, acc, cid, sid)


@functools.cache
def _sc_kernels():
    mesh = plsc.VectorSubcoreMesh(core_axis_name="c", subcore_axis_name="s",
                                  num_cores=NC, num_subcores=NS)
    params = pltpu.CompilerParams(use_tc_tiling_on_sc=False)
    sc_deg = pl.kernel(
        _sc_deg_body,
        out_type=jax.ShapeDtypeStruct((NC, N, D), jnp.float32),
        mesh=mesh,
        scratch_types=[
            pltpu.VMEM((JPW, BLK), jnp.int32),
            pltpu.VMEM((BLK, D), jnp.float32),
            pltpu.VMEM((RPS, D), jnp.float32),
            pltpu.SemaphoreType.DMA,
            pltpu.VMEM_SHARED((NP, D), jnp.float32),
        ],
        compiler_params=params,
    )
    sc_agg = pl.kernel(
        _sc_agg_body,
        out_type=jax.ShapeDtypeStruct((NC, N, D), jnp.float32),
        mesh=mesh,
        scratch_types=[
            pltpu.VMEM((NG, KB * BLK), jnp.int32),
            pltpu.VMEM((JPW, BLK), jnp.int32),
            pltpu.VMEM((2, KB * BLK, D), jnp.float32),
            pltpu.VMEM((RPS, D), jnp.float32),
            pltpu.SemaphoreType.DMA,
            pltpu.VMEM_SHARED((NP, D), jnp.float32),
        ],
        compiler_params=params,
    )
    return sc_deg, sc_agg


def _sc_agg_body(src_hbm, dst_hbm, hs_hbm, zeros_hbm, out_hbm, sidx_v, didx_v,
                 rows_v, stage_v, sem, acc):
    cid = lax.axis_index("c")
    sid = lax.axis_index("s")
    wid = cid * NS + sid
    _zero_acc(zeros_hbm, stage_v, acc, sid)
    pltpu.sync_copy(src_hbm.at[wid], sidx_v)
    pltpu.sync_copy(dst_hbm.at[wid], didx_v)
    plsc.subcore_barrier()

    def fire(g, p):
        pltpu.async_copy(hs_hbm.at[sidx_v.at[g]], rows_v.at[p], sem)

    fire(0, 0)

    def body(g, carry):
        p = lax.rem(g, 2)

        @pl.when(g < NG - 1)
        def _():
            fire(g + 1, 1 - p)

        pltpu.make_async_copy(hs_hbm.at[pl.ds(0, KB * BLK)],
                              rows_v.at[p], sem).wait()
        for t in range(KB):
            pltpu.sync_copy(rows_v.at[p, pl.ds(t * BLK, BLK)],
                            acc.at[didx_v.at[g * KB + t]], add=True)
        return carry

    lax.fori_loop(0, NG, body, 0)
    plsc.subcore_barrier()
    _copy_out(out_hbm, stage_v, acc, cid, sid)


def _tc_mm_body(x_ref, w_ref, o_ref):
    o_ref[...] = jnp.dot(x_ref[...], w_ref[...],
                         preferred_element_type=jnp.float32)


def _tc_scale_body(d0_ref, d1_ref, h_ref, hs_ref, dw_ref):
    deg = d0_ref[0, :, 0:1] + d1_ref[0, :, 0:1] + 1.0
    dis = lax.rsqrt(deg)
    dw_ref[...] = jnp.broadcast_to(dis, h_ref.shape)
    hs_ref[...] = dis * h_ref[...]


def _tc_mid_body(a0_ref, a1_ref, hs_ref, dw_ref, b_ref, w_ref, o_ref):
    dw = dw_ref[...]
    z = dw * (a0_ref[0] + a1_ref[0] + hs_ref[...]) + b_ref[...]
    z = jnp.maximum(z, 0.0)
    o_ref[...] = dw * jnp.dot(z, w_ref[...],
                              preferred_element_type=jnp.float32)


def _tc_fin_body(a0_ref, a1_ref, hs_ref, dw_ref, b_ref, o_ref):
    o_ref[...] = (dw_ref[...] * (a0_ref[0] + a1_ref[0] + hs_ref[...])
                  + b_ref[...])


def _row_spec(width):
    return pl.BlockSpec((RBS, width), lambda i: (i, 0))


def _part_spec(core):
    return pl.BlockSpec((1, RBS, D), lambda i, c=core: (c, i, 0))


def _full_spec(shape):
    return pl.BlockSpec(shape, lambda i: (0,) * len(shape))


def _nd_out():
    return jax.ShapeDtypeStruct((N, D), jnp.float32)


def kernel(x, edge_index, W1, b1, W2, b2):
    sc_deg, sc_agg = _sc_kernels()
    ei = edge_index.astype(jnp.int32)
    pad = EP - E
    src_r = jnp.concatenate(
        [ei[0], jnp.zeros((pad,), jnp.int32)]).reshape(NW, NG, KB * BLK)
    pad_dst = N + (jnp.arange(pad, dtype=jnp.int32) % (NP - N))
    dst_r = jnp.concatenate([ei[1], pad_dst]).reshape(NW, JPW, BLK)
    ones_rows = jnp.zeros((BLK, D), jnp.float32).at[:, 0].set(1.0)
    zeros_blk = jnp.zeros((RPS, D), jnp.float32)

    h1 = pl.pallas_call(
        _tc_mm_body,
        grid=(RB,),
        in_specs=[_row_spec(D_IN), _full_spec((D_IN, D))],
        out_specs=_row_spec(D),
        out_shape=_nd_out(),
    )(x, W1)

    degp = sc_deg(dst_r, ones_rows, zeros_blk)

    hs1, dw = pl.pallas_call(
        _tc_scale_body,
        grid=(RB,),
        in_specs=[_part_spec(0), _part_spec(1), _row_spec(D)],
        out_specs=[_row_spec(D), _row_spec(D)],
        out_shape=[_nd_out(), _nd_out()],
    )(degp, degp, h1)

    acc1 = sc_agg(src_r, dst_r, hs1, zeros_blk)

    hs2 = pl.pallas_call(
        _tc_mid_body,
        grid=(RB,),
        in_specs=[_part_spec(0), _part_spec(1), _row_spec(D), _row_spec(D),
                  _full_spec((1, D)), _full_spec((D, D))],
        out_specs=_row_spec(D),
        out_shape=_nd_out(),
    )(acc1, acc1, hs1, dw, b1.reshape(1, D), W2)

    acc2 = sc_agg(src_r, dst_r, hs2, zeros_blk)

    out = pl.pallas_call(
        _tc_fin_body,
        grid=(RB,),
        in_specs=[_part_spec(0), _part_spec(1), _row_spec(D), _row_spec(D),
                  _full_spec((1, D))],
        out_specs=_row_spec(D),
        out_shape=_nd_out(),
    )(acc2, acc2, hs2, dw, b2.reshape(1, D))
    return out


# speedup vs baseline: 1.4227x; 1.4026x over previous
import functools

import jax
import jax.numpy as jnp
from jax import lax
from jax.experimental import pallas as pl
from jax.experimental.pallas import tpu as pltpu
from jax.experimental.pallas import tpu_sc as plsc

N = 10000
E = 320000
D_IN = 128
D = 16
NC, NS = 2, 16
NW = NC * NS
BLK = 128
JPW = 80
KB = 8
NG = JPW // KB
EP = NW * JPW * BLK
NP = N + 112
RPS = 632
RPS_L = N - (NS - 1) * RPS
RB = 2
RBS = N // RB

def _zero_acc(zeros_hbm, stage_v, acc, sid):
    pltpu.sync_copy(zeros_hbm, stage_v)

    @pl.when(sid < NS - 1)
    def _():
        pltpu.sync_copy(stage_v, acc.at[pl.ds(sid * RPS, RPS)])

    @pl.when(sid == NS - 1)
    def _():
        pltpu.sync_copy(stage_v.at[pl.ds(0, RPS_L)],
                        acc.at[pl.ds(sid * RPS, RPS_L)])


def _copy_out(out_hbm, stage_v, acc, cid, sid):
    @pl.when(sid < NS - 1)
    def _():
        pltpu.sync_copy(acc.at[pl.ds(sid * RPS, RPS)], stage_v)
        pltpu.sync_copy(stage_v, out_hbm.at[cid, pl.ds(sid * RPS, RPS)])

    @pl.when(sid == NS - 1)
    def _():
        pltpu.sync_copy(acc.at[pl.ds(sid * RPS, RPS_L)],
                        stage_v.at[pl.ds(0, RPS_L)])
        pltpu.sync_copy(stage_v.at[pl.ds(0, RPS_L)],
                        out_hbm.at[cid, pl.ds(sid * RPS, RPS_L)])


def _sc_deg_body(dst_hbm, ones_hbm, zeros_hbm, out_hbm, idx_v, val_v, stage_v,
                 sem, acc):
    cid = lax.axis_index("c")
    sid = lax.axis_index("s")
    wid = cid * NS + sid
    _zero_acc(zeros_hbm, stage_v, acc, sid)
    pltpu.sync_copy(dst_hbm.at[wid], idx_v)
    pltpu.sync_copy(ones_hbm, val_v)
    plsc.subcore_barrier()

    def body(j, carry):
        pltpu.sync_copy(val_v, acc.at[idx_v.at[j]], add=True)
        return carry

    lax.fori_loop(0, JPW, body, 0)
    plsc.subcore_barrier()
    _copy_out(out_hbm, stage_v, acc, cid, sid)


@functools.cache
def _sc_kernels():
    mesh = plsc.VectorSubcoreMesh(core_axis_name="c", subcore_axis_name="s",
                                  num_cores=NC, num_subcores=NS)
    params = pltpu.CompilerParams(use_tc_tiling_on_sc=False)
    sc_deg = pl.kernel(
        _sc_deg_body,
        out_type=jax.ShapeDtypeStruct((NC, N, D), jnp.float32),
        mesh=mesh,
        scratch_types=[
            pltpu.VMEM((JPW, BLK), jnp.int32),
            pltpu.VMEM((BLK, D), jnp.float32),
            pltpu.VMEM((RPS, D), jnp.float32),
            pltpu.SemaphoreType.DMA,
            pltpu.VMEM_SHARED((NP, D), jnp.float32),
        ],
        compiler_params=params,
    )
    sc_agg = pl.kernel(
        _sc_agg_body,
        out_type=jax.ShapeDtypeStruct((NC, N, D), jnp.float32),
        mesh=mesh,
        scratch_types=[
            pltpu.VMEM((NG, KB * BLK), jnp.int32),
            pltpu.VMEM((JPW, BLK), jnp.int32),
            pltpu.VMEM((2, KB * BLK, D), jnp.float32),
            pltpu.VMEM((RPS, D), jnp.float32),
            pltpu.SemaphoreType.DMA,
            pltpu.VMEM_SHARED((NP, D), jnp.float32),
            pltpu.VMEM_SHARED((N, D), jnp.float32),
        ],
        compiler_params=params,
    )
    return sc_deg, sc_agg


def _sc_agg_body(src_hbm, dst_hbm, hs_hbm, zeros_hbm, out_hbm, sidx_v, didx_v,
                 rows_v, stage_v, sem, acc, hs_sh):
    cid = lax.axis_index("c")
    sid = lax.axis_index("s")
    wid = cid * NS + sid
    _zero_acc(zeros_hbm, stage_v, acc, sid)
    @pl.when(sid < NS - 1)
    def _():
        pltpu.sync_copy(hs_hbm.at[pl.ds(sid * RPS, RPS)], stage_v)
        pltpu.sync_copy(stage_v, hs_sh.at[pl.ds(sid * RPS, RPS)])

    @pl.when(sid == NS - 1)
    def _():
        pltpu.sync_copy(hs_hbm.at[pl.ds(sid * RPS, RPS_L)],
                        stage_v.at[pl.ds(0, RPS_L)])
        pltpu.sync_copy(stage_v.at[pl.ds(0, RPS_L)],
                        hs_sh.at[pl.ds(sid * RPS, RPS_L)])

    pltpu.sync_copy(src_hbm.at[wid], sidx_v)
    pltpu.sync_copy(dst_hbm.at[wid], didx_v)
    plsc.subcore_barrier()

    def fire(g, p):
        pltpu.async_copy(hs_sh.at[sidx_v.at[g]], rows_v.at[p], sem)

    fire(0, 0)

    def body(g, carry):
        p = lax.rem(g, 2)

        @pl.when(g < NG - 1)
        def _():
            fire(g + 1, 1 - p)

        pltpu.make_async_copy(hs_hbm.at[pl.ds(0, KB * BLK)],
                              rows_v.at[p], sem).wait()
        for t in range(KB):
            pltpu.sync_copy(rows_v.at[p, pl.ds(t * BLK, BLK)],
                            acc.at[didx_v.at[g * KB + t]], add=True)
        return carry

    lax.fori_loop(0, NG, body, 0)
    plsc.subcore_barrier()
    _copy_out(out_hbm, stage_v, acc, cid, sid)


def _tc_mm_body(x_ref, w_ref, o_ref):
    o_ref[...] = jnp.dot(x_ref[...], w_ref[...],
                         preferred_element_type=jnp.float32)


def _tc_scale_body(d0_ref, d1_ref, h_ref, hs_ref, dw_ref):
    deg = d0_ref[0, :, 0:1] + d1_ref[0, :, 0:1] + 1.0
    dis = lax.rsqrt(deg)
    dw_ref[...] = jnp.broadcast_to(dis, h_ref.shape)
    hs_ref[...] = dis * h_ref[...]


def _tc_mid_body(a0_ref, a1_ref, hs_ref, dw_ref, b_ref, w_ref, o_ref):
    dw = dw_ref[...]
    z = dw * (a0_ref[0] + a1_ref[0] + hs_ref[...]) + b_ref[...]
    z = jnp.maximum(z, 0.0)
    o_ref[...] = dw * jnp.dot(z, w_ref[...],
                              preferred_element_type=jnp.float32)


def _tc_fin_body(a0_ref, a1_ref, hs_ref, dw_ref, b_ref, o_ref):
    o_ref[...] = (dw_ref[...] * (a0_ref[0] + a1_ref[0] + hs_ref[...])
                  + b_ref[...])


def _row_spec(width):
    return pl.BlockSpec((RBS, width), lambda i: (i, 0))


def _part_spec(core):
    return pl.BlockSpec((1, RBS, D), lambda i, c=core: (c, i, 0))


def _full_spec(shape):
    return pl.BlockSpec(shape, lambda i: (0,) * len(shape))


def _nd_out():
    return jax.ShapeDtypeStruct((N, D), jnp.float32)


def kernel(x, edge_index, W1, b1, W2, b2):
    sc_deg, sc_agg = _sc_kernels()
    ei = edge_index.astype(jnp.int32)
    pad = EP - E
    src_r = jnp.concatenate(
        [ei[0], jnp.zeros((pad,), jnp.int32)]).reshape(NW, NG, KB * BLK)
    pad_dst = N + (jnp.arange(pad, dtype=jnp.int32) % (NP - N))
    dst_r = jnp.concatenate([ei[1], pad_dst]).reshape(NW, JPW, BLK)
    ones_rows = jnp.zeros((BLK, D), jnp.float32).at[:, 0].set(1.0)
    zeros_blk = jnp.zeros((RPS, D), jnp.float32)

    h1 = pl.pallas_call(
        _tc_mm_body,
        grid=(RB,),
        in_specs=[_row_spec(D_IN), _full_spec((D_IN, D))],
        out_specs=_row_spec(D),
        out_shape=_nd_out(),
    )(x, W1)

    degp = sc_deg(dst_r, ones_rows, zeros_blk)

    hs1, dw = pl.pallas_call(
        _tc_scale_body,
        grid=(RB,),
        in_specs=[_part_spec(0), _part_spec(1), _row_spec(D)],
        out_specs=[_row_spec(D), _row_spec(D)],
        out_shape=[_nd_out(), _nd_out()],
    )(degp, degp, h1)

    acc1 = sc_agg(src_r, dst_r, hs1, zeros_blk)

    hs2 = pl.pallas_call(
        _tc_mid_body,
        grid=(RB,),
        in_specs=[_part_spec(0), _part_spec(1), _row_spec(D), _row_spec(D),
                  _full_spec((1, D)), _full_spec((D, D))],
        out_specs=_row_spec(D),
        out_shape=_nd_out(),
    )(acc1, acc1, hs1, dw, b1.reshape(1, D), W2)

    acc2 = sc_agg(src_r, dst_r, hs2, zeros_blk)

    out = pl.pallas_call(
        _tc_fin_body,
        grid=(RB,),
        in_specs=[_part_spec(0), _part_spec(1), _row_spec(D), _row_spec(D),
                  _full_spec((1, D))],
        out_specs=_row_spec(D),
        out_shape=_nd_out(),
    )(acc2, acc2, hs2, dw, b2.reshape(1, D))
    return out


# speedup vs baseline: 1.4453x; 1.0159x over previous
import functools

import jax
import jax.numpy as jnp
from jax import lax
from jax.experimental import pallas as pl
from jax.experimental.pallas import tpu as pltpu
from jax.experimental.pallas import tpu_sc as plsc

N = 10000
E = 320000
D_IN = 128
D = 16
NC, NS = 2, 16
NW = NC * NS
BLK = 128
JPW = 80
KB = 8
NG = JPW // KB
EP = NW * JPW * BLK
NP = N + 112
RPS = 632
RPS_L = N - (NS - 1) * RPS
RB = 2
RBS = N // RB
WPE = JPW * BLK
GSZ = KB * BLK
DW = 8

def _zero_rows(zeros_hbm, stage_v, acc, sid):
    pltpu.sync_copy(zeros_hbm, stage_v)

    @pl.when(sid < NS - 1)
    def _():
        pltpu.sync_copy(stage_v, acc.at[pl.ds(sid * RPS, RPS)])

    @pl.when(sid == NS - 1)
    def _():
        pltpu.sync_copy(stage_v.at[pl.ds(0, RPS_L)],
                        acc.at[pl.ds(sid * RPS, RPS_L)])


def _copy_out_rows(out_hbm, stage_v, acc, cid, sid):
    @pl.when(sid < NS - 1)
    def _():
        pltpu.sync_copy(acc.at[pl.ds(sid * RPS, RPS)], stage_v)
        pltpu.sync_copy(stage_v, out_hbm.at[cid, pl.ds(sid * RPS, RPS)])

    @pl.when(sid == NS - 1)
    def _():
        pltpu.sync_copy(acc.at[pl.ds(sid * RPS, RPS_L)],
                        stage_v.at[pl.ds(0, RPS_L)])
        pltpu.sync_copy(stage_v.at[pl.ds(0, RPS_L)],
                        out_hbm.at[cid, pl.ds(sid * RPS, RPS_L)])


def _sc_deg_body(dst_hbm, ones_hbm, zeros_hbm, out_hbm, idx_v, val_v, stage_v,
                 sem, acc):
    cid = lax.axis_index("c")
    sid = lax.axis_index("s")
    wid = cid * NS + sid
    _zero_rows(zeros_hbm, stage_v, acc, sid)
    pltpu.sync_copy(dst_hbm.at[pl.ds(wid * WPE, WPE)], idx_v)
    pltpu.sync_copy(ones_hbm, val_v)
    plsc.subcore_barrier()

    def body(j, carry):
        pltpu.sync_copy(val_v, acc.at[idx_v.at[pl.ds(j * BLK, BLK)]],
                        add=True)
        return carry

    lax.fori_loop(0, JPW, body, 0)
    plsc.subcore_barrier()
    _copy_out_rows(out_hbm, stage_v, acc, cid, sid)


@functools.cache
def _sc_kernels():
    mesh = plsc.VectorSubcoreMesh(core_axis_name="c", subcore_axis_name="s",
                                  num_cores=NC, num_subcores=NS)
    params = pltpu.CompilerParams(use_tc_tiling_on_sc=False)
    sc_deg = pl.kernel(
        _sc_deg_body,
        out_type=jax.ShapeDtypeStruct((NC, N, DW), jnp.float32),
        mesh=mesh,
        scratch_types=[
            pltpu.VMEM((WPE,), jnp.int32),
            pltpu.VMEM((BLK, DW), jnp.float32),
            pltpu.VMEM((RPS, DW), jnp.float32),
            pltpu.SemaphoreType.DMA,
            pltpu.VMEM_SHARED((NP, DW), jnp.float32),
        ],
        compiler_params=params,
    )
    sc_agg = pl.kernel(
        _sc_agg_body,
        out_type=jax.ShapeDtypeStruct((NC, N, D), jnp.float32),
        mesh=mesh,
        scratch_types=[
            pltpu.VMEM((WPE,), jnp.int32),
            pltpu.VMEM((WPE,), jnp.int32),
            pltpu.VMEM((2, GSZ, D), jnp.float32),
            pltpu.VMEM((RPS, D), jnp.float32),
            pltpu.SemaphoreType.DMA,
            pltpu.VMEM_SHARED((NP, D), jnp.float32),
            pltpu.VMEM_SHARED((N, D), jnp.float32),
        ],
        compiler_params=params,
    )
    return sc_deg, sc_agg


def _sc_agg_body(src_hbm, dst_hbm, hs_hbm, zeros_hbm, out_hbm, sidx_v, didx_v,
                 rows_v, stage_v, sem, acc, hs_sh):
    cid = lax.axis_index("c")
    sid = lax.axis_index("s")
    wid = cid * NS + sid
    _zero_rows(zeros_hbm, stage_v, acc, sid)
    @pl.when(sid < NS - 1)
    def _():
        pltpu.sync_copy(hs_hbm.at[pl.ds(sid * RPS, RPS)], stage_v)
        pltpu.sync_copy(stage_v, hs_sh.at[pl.ds(sid * RPS, RPS)])

    @pl.when(sid == NS - 1)
    def _():
        pltpu.sync_copy(hs_hbm.at[pl.ds(sid * RPS, RPS_L)],
                        stage_v.at[pl.ds(0, RPS_L)])
        pltpu.sync_copy(stage_v.at[pl.ds(0, RPS_L)],
                        hs_sh.at[pl.ds(sid * RPS, RPS_L)])

    pltpu.sync_copy(src_hbm.at[pl.ds(wid * WPE, WPE)], sidx_v)
    pltpu.sync_copy(dst_hbm.at[pl.ds(wid * WPE, WPE)], didx_v)
    plsc.subcore_barrier()

    def fire(g, p):
        pltpu.async_copy(hs_sh.at[sidx_v.at[pl.ds(g * GSZ, GSZ)]],
                         rows_v.at[p], sem)

    fire(0, 0)

    def body(g, carry):
        p = lax.rem(g, 2)

        @pl.when(g < NG - 1)
        def _():
            fire(g + 1, 1 - p)

        pltpu.make_async_copy(hs_hbm.at[pl.ds(0, KB * BLK)],
                              rows_v.at[p], sem).wait()
        for t in range(KB):
            pltpu.sync_copy(rows_v.at[p, pl.ds(t * BLK, BLK)],
                            acc.at[didx_v.at[pl.ds((g * KB + t) * BLK, BLK)]],
                            add=True)
        return carry

    lax.fori_loop(0, NG, body, 0)
    plsc.subcore_barrier()
    _copy_out_rows(out_hbm, stage_v, acc, cid, sid)


def _tc_mm_body(x_ref, w_ref, o_ref):
    o_ref[...] = jnp.dot(x_ref[...], w_ref[...],
                         preferred_element_type=jnp.float32)


def _tc_scale_body(d0_ref, d1_ref, h_ref, hs_ref, dw_ref):
    deg = d0_ref[0, :, 0:1] + d1_ref[0, :, 0:1] + 1.0
    dis = lax.rsqrt(deg)
    dw_ref[...] = jnp.broadcast_to(dis, h_ref.shape)
    hs_ref[...] = dis * h_ref[...]


def _tc_mid_body(a0_ref, a1_ref, hs_ref, dw_ref, b_ref, w_ref, o_ref):
    dw = dw_ref[...]
    z = dw * (a0_ref[0] + a1_ref[0] + hs_ref[...]) + b_ref[...]
    z = jnp.maximum(z, 0.0)
    o_ref[...] = dw * jnp.dot(z, w_ref[...],
                              preferred_element_type=jnp.float32)


def _tc_fin_body(a0_ref, a1_ref, hs_ref, dw_ref, b_ref, o_ref):
    o_ref[...] = (dw_ref[...] * (a0_ref[0] + a1_ref[0] + hs_ref[...])
                  + b_ref[...])


def _row_spec(width):
    return pl.BlockSpec((RBS, width), lambda i: (i, 0))


def _part_spec(core, width=D):
    return pl.BlockSpec((1, RBS, width), lambda i, c=core: (c, i, 0))


def _full_spec(shape):
    return pl.BlockSpec(shape, lambda i: (0,) * len(shape))


def _nd_out():
    return jax.ShapeDtypeStruct((N, D), jnp.float32)


def kernel(x, edge_index, W1, b1, W2, b2):
    sc_deg, sc_agg = _sc_kernels()
    ei = edge_index.astype(jnp.int32)
    pad = EP - E
    src_r = jnp.concatenate([ei[0], jnp.zeros((pad,), jnp.int32)])
    pad_dst = N + (jnp.arange(pad, dtype=jnp.int32) % (NP - N))
    dst_r = jnp.concatenate([ei[1], pad_dst])
    ones_rows = jnp.zeros((BLK, DW), jnp.float32).at[:, 0].set(1.0)
    zeros_deg = jnp.zeros((RPS, DW), jnp.float32)
    zeros_blk = jnp.zeros((RPS, D), jnp.float32)

    h1 = pl.pallas_call(
        _tc_mm_body,
        grid=(RB,),
        in_specs=[_row_spec(D_IN), _full_spec((D_IN, D))],
        out_specs=_row_spec(D),
        out_shape=_nd_out(),
    )(x, W1)

    degp = sc_deg(dst_r, ones_rows, zeros_deg)

    hs1, dw = pl.pallas_call(
        _tc_scale_body,
        grid=(RB,),
        in_specs=[_part_spec(0, DW), _part_spec(1, DW), _row_spec(D)],
        out_specs=[_row_spec(D), _row_spec(D)],
        out_shape=[_nd_out(), _nd_out()],
    )(degp, degp, h1)

    acc1 = sc_agg(src_r, dst_r, hs1, zeros_blk)

    hs2 = pl.pallas_call(
        _tc_mid_body,
        grid=(RB,),
        in_specs=[_part_spec(0), _part_spec(1), _row_spec(D), _row_spec(D),
                  _full_spec((1, D)), _full_spec((D, D))],
        out_specs=_row_spec(D),
        out_shape=_nd_out(),
    )(acc1, acc1, hs1, dw, b1.reshape(1, D), W2)

    acc2 = sc_agg(src_r, dst_r, hs2, zeros_blk)

    out = pl.pallas_call(
        _tc_fin_body,
        grid=(RB,),
        in_specs=[_part_spec(0), _part_spec(1), _row_spec(D), _row_spec(D),
                  _full_spec((1, D))],
        out_specs=_row_spec(D),
        out_shape=_nd_out(),
    )(acc2, acc2, hs2, dw, b2.reshape(1, D))
    return out
